# baseline (speedup 1.0000x reference)
import jax
import jax.numpy as jnp
from jax.experimental import pallas as pl

_BLOCK_ROWS = 256


def _rowscale_kernel(x_ref, s_ref, o_ref):
    s = jnp.sum(s_ref[...], axis=1, keepdims=True)
    o_ref[...] = x_ref[...] * s


def kernel(x, top_scores, selected_experts_indices, num_tokens_per_expert):
    del selected_experts_indices, num_tokens_per_expert
    n, d = x.shape
    k = top_scores.shape[1]
    grid = (n // _BLOCK_ROWS,)
    return pl.pallas_call(
        _rowscale_kernel,
        grid=grid,
        in_specs=[
            pl.BlockSpec((_BLOCK_ROWS, d), lambda i: (i, 0)),
            pl.BlockSpec((_BLOCK_ROWS, k), lambda i: (i, 0)),
        ],
        out_specs=pl.BlockSpec((_BLOCK_ROWS, d), lambda i: (i, 0)),
        out_shape=jax.ShapeDtypeStruct((n, d), x.dtype),
    )(x, top_scores)


# speedup vs baseline: 1.0142x; 1.0142x over previous
import jax
import jax.numpy as jnp
from jax.experimental import pallas as pl
from jax.experimental.pallas import tpu as pltpu

_BLOCK_ROWS = 512


def _rowscale_kernel(x_ref, s_ref, o_ref):
    s = jnp.sum(s_ref[...], axis=1, keepdims=True)
    o_ref[...] = x_ref[...] * s


def kernel(x, top_scores, selected_experts_indices, num_tokens_per_expert):
    del selected_experts_indices, num_tokens_per_expert
    n, d = x.shape
    k = top_scores.shape[1]
    grid = (n // _BLOCK_ROWS,)
    return pl.pallas_call(
        _rowscale_kernel,
        grid=grid,
        in_specs=[
            pl.BlockSpec((_BLOCK_ROWS, d), lambda i: (i, 0)),
            pl.BlockSpec((_BLOCK_ROWS, k), lambda i: (i, 0)),
        ],
        out_specs=pl.BlockSpec((_BLOCK_ROWS, d), lambda i: (i, 0)),
        out_shape=jax.ShapeDtypeStruct((n, d), x.dtype),
        compiler_params=pltpu.CompilerParams(
            dimension_semantics=("parallel",),
        ),
    )(x, top_scores)
